# Initial kernel scaffold; baseline (speedup 1.0000x reference)
import jax
import jax.numpy as jnp
from jax.experimental import pallas as pl


def kernel(input_ids, embed_table):
    raise NotImplementedError("write your pallas kernel here")


# speedup vs baseline: 1.6349x; 1.6349x over previous
import functools

import jax
import jax.numpy as jnp
from jax import lax
from jax.experimental import pallas as pl
from jax.experimental.pallas import tpu as pltpu
from jax.experimental.pallas import tpu_sc as plsc

_NC = 2
_NS = 16
_NW = _NC * _NS
_C = 32


def _embed_sc(ids3, table):
    nw, nchunk, c = ids3.shape
    total = nw * nchunk * c
    d = table.shape[1]
    mesh = plsc.VectorSubcoreMesh(
        core_axis_name="c", subcore_axis_name="s",
        num_cores=_NC, num_subcores=_NS)

    @functools.partial(
        pl.kernel,
        out_type=jax.ShapeDtypeStruct((total, d), jnp.float32),
        mesh=mesh,
        scratch_types=[
            pltpu.VMEM((nchunk, c), jnp.int32),
            pltpu.VMEM((c, d), jnp.float32),
            pltpu.VMEM((c, d), jnp.float32),
            pltpu.SemaphoreType.DMA,
            pltpu.SemaphoreType.DMA,
        ],
    )
    def k(ids_hbm, table_hbm, out_hbm, idx_v, buf0, buf1, sem0, sem1):
        wid = lax.axis_index("s") * _NC + lax.axis_index("c")
        base = wid * (nchunk * c)
        pltpu.sync_copy(ids_hbm.at[wid], idx_v)
        bufs = (buf0, buf1)
        sems = (sem0, sem1)
        cps = [None, None]
        cps[0] = pltpu.async_copy(table_hbm.at[idx_v.at[0]], buf0, sem0)
        for j in range(nchunk):
            cur = j % 2
            nxt = 1 - cur
            if j + 1 < nchunk:
                cps[nxt] = pltpu.async_copy(
                    table_hbm.at[idx_v.at[j + 1]], bufs[nxt], sems[nxt])
            cps[cur].wait()
            pltpu.sync_copy(bufs[cur], out_hbm.at[pl.ds(base + j * c, c)])

    return k(ids3, table)


def kernel(input_ids, embed_table):
    b, s = input_ids.shape
    d = embed_table.shape[1]
    total = b * s
    nchunk = total // (_NW * _C)
    ids3 = input_ids.reshape(_NW, nchunk, _C).astype(jnp.int32)
    out = _embed_sc(ids3, embed_table.astype(jnp.float32))
    return out.reshape(b, s, d)


# speedup vs baseline: 1.6576x; 1.0139x over previous
import functools

import jax
import jax.numpy as jnp
from jax import lax
from jax.experimental import pallas as pl
from jax.experimental.pallas import tpu as pltpu
from jax.experimental.pallas import tpu_sc as plsc

_NC = 2
_NS = 16
_NW = _NC * _NS
_C = 32


def _embed_sc(ids3, table):
    nw, nchunk, c = ids3.shape
    total = nw * nchunk * c
    d = table.shape[1]
    mesh = plsc.VectorSubcoreMesh(
        core_axis_name="c", subcore_axis_name="s",
        num_cores=_NC, num_subcores=_NS)

    @functools.partial(
        pl.kernel,
        out_type=jax.ShapeDtypeStruct((total, d), jnp.float32),
        mesh=mesh,
        scratch_types=[
            pltpu.VMEM((nchunk, c), jnp.int32),
            pltpu.VMEM((c, d), jnp.float32),
            pltpu.VMEM((c, d), jnp.float32),
            pltpu.VMEM((c, d), jnp.float32),
            pltpu.SemaphoreType.DMA,
            pltpu.SemaphoreType.DMA,
            pltpu.SemaphoreType.DMA,
        ],
    )
    def k(ids_hbm, table_hbm, out_hbm, idx_v,
          buf0, buf1, buf2, sem0, sem1, sem2):
        wid = lax.axis_index("s") * _NC + lax.axis_index("c")
        base = wid * (nchunk * c)
        pltpu.sync_copy(ids_hbm.at[wid], idx_v)
        nbuf = 3
        bufs = (buf0, buf1, buf2)
        sems = (sem0, sem1, sem2)
        cps = [None] * nbuf
        for j in range(nbuf - 1):
            cps[j] = pltpu.async_copy(
                table_hbm.at[idx_v.at[j]], bufs[j], sems[j])
        for j in range(nchunk):
            cur = j % nbuf
            if j + nbuf - 1 < nchunk:
                nxt = (j + nbuf - 1) % nbuf
                cps[nxt] = pltpu.async_copy(
                    table_hbm.at[idx_v.at[j + nbuf - 1]], bufs[nxt], sems[nxt])
            cps[cur].wait()
            pltpu.sync_copy(bufs[cur], out_hbm.at[pl.ds(base + j * c, c)])

    return k(ids3, table)


def kernel(input_ids, embed_table):
    b, s = input_ids.shape
    d = embed_table.shape[1]
    total = b * s
    nchunk = total // (_NW * _C)
    ids3 = input_ids.reshape(_NW, nchunk, _C).astype(jnp.int32)
    out = _embed_sc(ids3, embed_table.astype(jnp.float32))
    return out.reshape(b, s, d)
